# baseline (speedup 1.0000x reference)
import functools

import jax
import jax.numpy as jnp
from jax import lax
from jax.experimental import pallas as pl
from jax.experimental.pallas import tpu as pltpu, tpu_sc as plsc

NC, NS, L = 2, 16, 16
NW = NC * NS

CH = 512
SUB = 128
NSUB = CH // SUB
STEPS = CH // L
RB = 8


def _ffloor(z):
    zi = z.astype(jnp.int32)
    zf = zi.astype(jnp.float32)
    return jnp.where(zf > z, zf - 1.0, zf)


def _ifloor(z):
    zi = z.astype(jnp.int32)
    return jnp.where(zi.astype(jnp.float32) > z, zi - 1, zi)


def _relayout_body(x_ref, o_ref):
    Wn = x_ref.shape[1]
    for r in range(RB):
        o_ref[pl.ds(r * Wn, Wn), :] = x_ref[r]


def _relayout(sat):
    Hn, Wn, C = sat.shape
    return pl.pallas_call(
        _relayout_body,
        grid=(Hn // RB,),
        in_specs=[pl.BlockSpec((RB, Wn, C), lambda g: (g, 0, 0))],
        out_specs=pl.BlockSpec((RB * Wn, C), lambda g: (g, 0)),
        out_shape=jax.ShapeDtypeStruct((Hn * Wn, C), jnp.float32),
    )(sat)


def _sc_body(Hn, Wn, C, QW,
             x_hbm, satf_hbm, out_hbm,
             xb0, xb1,
             i00a, i01a, i10a, i11a, i00b, i01b, i10b, i11b,
             sta, stb, swa, swb,
             g00a, g01a, g10a, g11a, g00b, g01b, g10b, g11b,
             ob0, ob1, lrv, lcv, lcidx, cornerv,
             gsem0, gsem1, xsem0, xsem1, osem0, osem1):
    wid = lax.axis_index("s") * NC + lax.axis_index("c")
    base = wid * QW
    NCH = QW // CH

    xb = (xb0, xb1)
    idx = ((i00a, i01a, i10a, i11a), (i00b, i01b, i10b, i11b))
    sti = (sta, stb)
    stw = (swa, swb)
    gb = ((g00a, g01a, g10a, g11a), (g00b, g01b, g10b, g11b))
    ob = (ob0, ob1)
    gsem = (gsem0, gsem1)
    xsem = (xsem0, xsem1)
    osem = (osem0, osem1)

    iota = lax.iota(jnp.int32, L)
    fH = float(Hn)
    fW = float(Wn)
    lim = Hn * Wn - 1

    pltpu.sync_copy(satf_hbm.at[pl.ds((Hn - 1) * Wn, Wn)], lrv)

    def lcidx_step(s, carry):
        vec = (s * L + iota) * Wn + (Wn - 1)
        lcidx[s // (SUB // L), pl.ds((s % (SUB // L)) * L, L)] = vec
        return carry

    lax.fori_loop(0, Hn // L, lcidx_step, 0)
    lcdescs = [pltpu.async_copy(satf_hbm.at[lcidx.at[j]],
                                lcv.at[pl.ds(j * SUB, SUB)], gsem0)
               for j in range(Hn // SUB)]
    for dsc in lcdescs:
        dsc.wait()
    for ch in range(C):
        cornerv[ch] = plsc.load_gather(
            lrv, [jnp.full((L,), Wn - 1, jnp.int32),
                  jnp.full((L,), ch, jnp.int32)])

    def phase_a(p):
        xbuf = xb[p]
        i0, i1, i2, i3 = idx[p]
        st = sti[p]
        sw = stw[p]

        def stepA(s, carry2):
            q16 = s * L + iota
            z0 = jnp.zeros((L,), jnp.int32)
            cu = plsc.load_gather(xbuf, [q16, z0])
            cv = plsc.load_gather(xbuf, [q16, z0 + 1])
            d0 = plsc.load_gather(xbuf, [q16, z0 + 2])
            d1 = plsc.load_gather(xbuf, [q16, z0 + 3])
            s0 = cu - d0 * 0.5
            e0 = cu + d0 * 0.5
            s1 = cv - d1 * 0.5
            e1 = cv + d1 * 0.5
            ns0 = jnp.minimum(s0, e0)
            ne0 = jnp.maximum(s0, e0)
            ns1 = jnp.minimum(s1, e1)
            ne1 = jnp.maximum(s1, e1)
            ns0 = ns0 - _ffloor(ns0)
            ne0 = ne0 - _ffloor(ne0)
            ns1 = ns1 - _ffloor(ns1)
            ne1 = ne1 - _ffloor(ne1)
            ix0 = _ifloor(fH * ne0 - 0.5)
            ix1 = _ifloor(fH * ns0 - 1.5)
            iy0 = _ifloor(fW * ne1 - 0.5)
            iy1 = _ifloor(fW * ns1 - 1.5)
            wx = jnp.where(ns0 > ne0, 1.0, 0.0)
            wy = jnp.where(ns1 > ne1, 1.0, 0.0)

            f00 = jnp.clip(ix0 * Wn + iy0, 0, lim)
            f01 = jnp.clip(ix0 * Wn + iy1, 0, lim)
            f10 = jnp.clip(ix1 * Wn + iy0, 0, lim)
            f11 = jnp.clip(ix1 * Wn + iy1, 0, lim)

            blk = s // (SUB // L)
            off = (s % (SUB // L)) * L
            i0[blk, pl.ds(off, L)] = f00
            i1[blk, pl.ds(off, L)] = f01
            i2[blk, pl.ds(off, L)] = f10
            i3[blk, pl.ds(off, L)] = f11

            sl = pl.ds(s * L, L)
            st[0, sl] = ix0
            st[1, sl] = ix1
            st[2, sl] = iy0
            st[3, sl] = iy1
            sw[0, sl] = wx
            sw[1, sl] = wy
            return carry2

        lax.fori_loop(0, STEPS, stepA, 0)

    def fire_gathers(cc, p):
        descs = []
        for idxr, gbuf in zip(idx[p], gb[p]):
            for j in range(NSUB):
                descs.append(pltpu.async_copy(
                    satf_hbm.at[idxr.at[j]], gbuf.at[pl.ds(j * SUB, SUB)],
                    gsem[p]))
        return descs

    def phase_b(p):
        b00, b01, b10, b11 = gb[p]
        st = sti[p]
        sw = stw[p]
        obuf = ob[p]

        def stepB(s, carry2):
            q16 = s * L + iota
            sl = pl.ds(s * L, L)
            ix0 = st[0, sl]
            ix1 = st[1, sl]
            iy0 = st[2, sl]
            iy1 = st[3, sl]
            wx = sw[0, sl]
            wy = sw[1, sl]

            one = jnp.ones((L,), jnp.float32)
            zero = jnp.zeros((L,), jnp.float32)
            mx0 = jnp.where((ix0 >= 0) & (ix0 < Hn), one, zero)
            mx1 = jnp.where((ix1 >= 0) & (ix1 < Hn), one, zero)
            my0 = jnp.where((iy0 >= 0) & (iy0 < Wn), one, zero)
            my1 = jnp.where((iy1 >= 0) & (iy1 < Wn), one, zero)

            c00 = mx0 * my0
            c01 = -(mx0 * my1)
            c10 = -(mx1 * my0)
            c11 = mx1 * my1
            cr0 = wx * my0
            cr1 = -(wx * my1)
            cc0 = mx0 * wy
            cc1 = -(mx1 * wy)
            ccc = wx * wy

            jy0 = jnp.clip(iy0, 0, Wn - 1)
            jy1 = jnp.clip(iy1, 0, Wn - 1)
            jx0 = jnp.clip(ix0, 0, Hn - 1)
            jx1 = jnp.clip(ix1, 0, Hn - 1)

            for ch in range(C):
                chv = jnp.full((L,), ch, jnp.int32)
                acc = cornerv[ch] * ccc
                acc = acc + c00 * plsc.load_gather(b00, [q16, chv])
                acc = acc + c01 * plsc.load_gather(b01, [q16, chv])
                acc = acc + c10 * plsc.load_gather(b10, [q16, chv])
                acc = acc + c11 * plsc.load_gather(b11, [q16, chv])
                acc = acc + cr0 * plsc.load_gather(lrv, [jy0, chv])
                acc = acc + cr1 * plsc.load_gather(lrv, [jy1, chv])
                acc = acc + cc0 * plsc.load_gather(lcv, [jx0, chv])
                acc = acc + cc1 * plsc.load_gather(lcv, [jx1, chv])
                plsc.store_scatter(obuf, [q16, chv], acc)
            return carry2

        lax.fori_loop(0, STEPS, stepB, 0)

    gdescs = [None] * NCH
    xdescs = [None] * NCH
    odescs = [None] * NCH

    pltpu.sync_copy(x_hbm.at[pl.ds(base, CH)], xb[0])
    if NCH > 1:
        xdescs[1] = pltpu.async_copy(
            x_hbm.at[pl.ds(base + CH, CH)], xb[1], xsem[1])
    phase_a(0)
    gdescs[0] = fire_gathers(0, 0)

    for cc in range(1, NCH):
        p = cc & 1
        if cc + 1 < NCH:
            xdescs[cc + 1] = pltpu.async_copy(
                x_hbm.at[pl.ds(base + (cc + 1) * CH, CH)], xb[1 - p],
                xsem[1 - p])
        xdescs[cc].wait()
        phase_a(p)
        gdescs[cc] = fire_gathers(cc, p)

        for dsc in gdescs[cc - 1]:
            dsc.wait()
        if cc - 3 >= 0:
            odescs[cc - 3].wait()
        phase_b(1 - p)
        odescs[cc - 1] = pltpu.async_copy(
            ob[1 - p], out_hbm.at[pl.ds(base + (cc - 1) * CH, CH)],
            osem[1 - p])

    pl_ = (NCH - 1) & 1
    for dsc in gdescs[NCH - 1]:
        dsc.wait()
    if NCH - 3 >= 0:
        odescs[NCH - 3].wait()
    phase_b(pl_)
    odescs[NCH - 1] = pltpu.async_copy(
        ob[pl_], out_hbm.at[pl.ds(base + (NCH - 1) * CH, CH)], osem[pl_])
    if NCH - 2 >= 0:
        odescs[NCH - 2].wait()
    odescs[NCH - 1].wait()


def kernel(x, sat):
    N = x.shape[0]
    Hn, Wn, C = sat.shape
    QW = N // NW

    satf = _relayout(sat)

    mesh = plsc.VectorSubcoreMesh(core_axis_name="c", subcore_axis_name="s",
                                  num_cores=NC, num_subcores=NS)
    body = functools.partial(_sc_body, Hn, Wn, C, QW)
    fn = pl.kernel(
        body,
        out_type=jax.ShapeDtypeStruct((N, C), jnp.float32),
        mesh=mesh,
        compiler_params=pltpu.CompilerParams(needs_layout_passes=False,
                                             use_tc_tiling_on_sc=False),
        scratch_types=[
            pltpu.VMEM((CH, 4), jnp.float32),
            pltpu.VMEM((CH, 4), jnp.float32),
            pltpu.VMEM((NSUB, SUB), jnp.int32),
            pltpu.VMEM((NSUB, SUB), jnp.int32),
            pltpu.VMEM((NSUB, SUB), jnp.int32),
            pltpu.VMEM((NSUB, SUB), jnp.int32),
            pltpu.VMEM((NSUB, SUB), jnp.int32),
            pltpu.VMEM((NSUB, SUB), jnp.int32),
            pltpu.VMEM((NSUB, SUB), jnp.int32),
            pltpu.VMEM((NSUB, SUB), jnp.int32),
            pltpu.VMEM((4, CH), jnp.int32),
            pltpu.VMEM((4, CH), jnp.int32),
            pltpu.VMEM((2, CH), jnp.float32),
            pltpu.VMEM((2, CH), jnp.float32),
            pltpu.VMEM((CH, C), jnp.float32),
            pltpu.VMEM((CH, C), jnp.float32),
            pltpu.VMEM((CH, C), jnp.float32),
            pltpu.VMEM((CH, C), jnp.float32),
            pltpu.VMEM((CH, C), jnp.float32),
            pltpu.VMEM((CH, C), jnp.float32),
            pltpu.VMEM((CH, C), jnp.float32),
            pltpu.VMEM((CH, C), jnp.float32),
            pltpu.VMEM((CH, C), jnp.float32),
            pltpu.VMEM((CH, C), jnp.float32),
            pltpu.VMEM((Wn, C), jnp.float32),
            pltpu.VMEM((Hn, C), jnp.float32),
            pltpu.VMEM((Hn // SUB, SUB), jnp.int32),
            pltpu.VMEM((C, L), jnp.float32),
            pltpu.SemaphoreType.DMA,
            pltpu.SemaphoreType.DMA,
            pltpu.SemaphoreType.DMA,
            pltpu.SemaphoreType.DMA,
            pltpu.SemaphoreType.DMA,
            pltpu.SemaphoreType.DMA,
        ],
    )
    return fn(x, satf)


# speedup vs baseline: 4.6101x; 4.6101x over previous
import functools

import jax
import jax.numpy as jnp
from jax import lax
from jax.experimental import pallas as pl
from jax.experimental.pallas import tpu as pltpu, tpu_sc as plsc

NC, NS, L = 2, 16, 16
NW = NC * NS

CH = 512
SUB = 128
NSUB = CH // SUB
STEPS = CH // L
RB = 8


def _ffloor(z):
    zi = z.astype(jnp.int32)
    zf = zi.astype(jnp.float32)
    return jnp.where(zf > z, zf - 1.0, zf)


def _ifloor(z):
    zi = z.astype(jnp.int32)
    return jnp.where(zi.astype(jnp.float32) > z, zi - 1, zi)


def _relayout_body(x_ref, o_ref):
    o_ref[...] = x_ref[...].reshape(o_ref.shape)


def _relayout(sat):
    Hn, Wn, C = sat.shape
    satw = sat.reshape(Hn, Wn * C)
    kb = RB * Wn * C // 1024
    outp = pl.pallas_call(
        _relayout_body,
        grid=(Hn // RB,),
        in_specs=[pl.BlockSpec((RB, Wn * C), lambda g: (g, 0))],
        out_specs=pl.BlockSpec((kb, 8, 128), lambda g: (g, 0, 0)),
        out_shape=jax.ShapeDtypeStruct((Hn * Wn * C // 1024, 8, 128),
                                       jnp.float32),
    )(satw)
    return outp.reshape(Hn * Wn, C)


def _sc_body(Hn, Wn, C, QW,
             x_hbm, satf_hbm, out_hbm,
             xb0, xb1,
             i00a, i01a, i10a, i11a, i00b, i01b, i10b, i11b,
             sta, stb, swa, swb,
             g00a, g01a, g10a, g11a, g00b, g01b, g10b, g11b,
             ob0, ob1, lrv, lcv, lcidx, cornerv,
             gsem0, gsem1, xsem0, xsem1, osem0, osem1):
    wid = lax.axis_index("s") * NC + lax.axis_index("c")
    base = wid * QW
    NCH = QW // CH

    xb = (xb0, xb1)
    idx = ((i00a, i01a, i10a, i11a), (i00b, i01b, i10b, i11b))
    sti = (sta, stb)
    stw = (swa, swb)
    gb = ((g00a, g01a, g10a, g11a), (g00b, g01b, g10b, g11b))
    ob = (ob0, ob1)
    gsem = (gsem0, gsem1)
    xsem = (xsem0, xsem1)
    osem = (osem0, osem1)

    iota = lax.iota(jnp.int32, L)
    fH = float(Hn)
    fW = float(Wn)
    lim = Hn * Wn - 1

    pltpu.sync_copy(satf_hbm.at[pl.ds((Hn - 1) * Wn, Wn)], lrv)

    def lcidx_step(s, carry):
        vec = (s * L + iota) * Wn + (Wn - 1)
        lcidx[s // (SUB // L), pl.ds((s % (SUB // L)) * L, L)] = vec
        return carry

    lax.fori_loop(0, Hn // L, lcidx_step, 0)
    lcdescs = [pltpu.async_copy(satf_hbm.at[lcidx.at[j]],
                                lcv.at[pl.ds(j * SUB, SUB)], gsem0)
               for j in range(Hn // SUB)]
    for dsc in lcdescs:
        dsc.wait()
    for ch in range(C):
        cornerv[ch] = plsc.load_gather(
            lrv, [jnp.full((L,), Wn - 1, jnp.int32),
                  jnp.full((L,), ch, jnp.int32)])

    def phase_a(p):
        xbuf = xb[p]
        i0, i1, i2, i3 = idx[p]
        st = sti[p]
        sw = stw[p]

        def stepA(s, carry2):
            q16 = s * L + iota
            z0 = jnp.zeros((L,), jnp.int32)
            cu = plsc.load_gather(xbuf, [q16, z0])
            cv = plsc.load_gather(xbuf, [q16, z0 + 1])
            d0 = plsc.load_gather(xbuf, [q16, z0 + 2])
            d1 = plsc.load_gather(xbuf, [q16, z0 + 3])
            s0 = cu - d0 * 0.5
            e0 = cu + d0 * 0.5
            s1 = cv - d1 * 0.5
            e1 = cv + d1 * 0.5
            ns0 = jnp.minimum(s0, e0)
            ne0 = jnp.maximum(s0, e0)
            ns1 = jnp.minimum(s1, e1)
            ne1 = jnp.maximum(s1, e1)
            ns0 = ns0 - _ffloor(ns0)
            ne0 = ne0 - _ffloor(ne0)
            ns1 = ns1 - _ffloor(ns1)
            ne1 = ne1 - _ffloor(ne1)
            ix0 = _ifloor(fH * ne0 - 0.5)
            ix1 = _ifloor(fH * ns0 - 1.5)
            iy0 = _ifloor(fW * ne1 - 0.5)
            iy1 = _ifloor(fW * ns1 - 1.5)
            wx = jnp.where(ns0 > ne0, 1.0, 0.0)
            wy = jnp.where(ns1 > ne1, 1.0, 0.0)

            f00 = jnp.clip(ix0 * Wn + iy0, 0, lim)
            f01 = jnp.clip(ix0 * Wn + iy1, 0, lim)
            f10 = jnp.clip(ix1 * Wn + iy0, 0, lim)
            f11 = jnp.clip(ix1 * Wn + iy1, 0, lim)

            blk = s // (SUB // L)
            off = (s % (SUB // L)) * L
            i0[blk, pl.ds(off, L)] = f00
            i1[blk, pl.ds(off, L)] = f01
            i2[blk, pl.ds(off, L)] = f10
            i3[blk, pl.ds(off, L)] = f11

            sl = pl.ds(s * L, L)
            st[0, sl] = ix0
            st[1, sl] = ix1
            st[2, sl] = iy0
            st[3, sl] = iy1
            sw[0, sl] = wx
            sw[1, sl] = wy
            return carry2

        lax.fori_loop(0, STEPS, stepA, 0)

    def fire_gathers(cc, p):
        descs = []
        for idxr, gbuf in zip(idx[p], gb[p]):
            for j in range(NSUB):
                descs.append(pltpu.async_copy(
                    satf_hbm.at[idxr.at[j]], gbuf.at[pl.ds(j * SUB, SUB)],
                    gsem[p]))
        return descs

    def phase_b(p):
        b00, b01, b10, b11 = gb[p]
        st = sti[p]
        sw = stw[p]
        obuf = ob[p]

        def stepB(s, carry2):
            q16 = s * L + iota
            sl = pl.ds(s * L, L)
            ix0 = st[0, sl]
            ix1 = st[1, sl]
            iy0 = st[2, sl]
            iy1 = st[3, sl]
            wx = sw[0, sl]
            wy = sw[1, sl]

            one = jnp.ones((L,), jnp.float32)
            zero = jnp.zeros((L,), jnp.float32)
            mx0 = jnp.where((ix0 >= 0) & (ix0 < Hn), one, zero)
            mx1 = jnp.where((ix1 >= 0) & (ix1 < Hn), one, zero)
            my0 = jnp.where((iy0 >= 0) & (iy0 < Wn), one, zero)
            my1 = jnp.where((iy1 >= 0) & (iy1 < Wn), one, zero)

            c00 = mx0 * my0
            c01 = -(mx0 * my1)
            c10 = -(mx1 * my0)
            c11 = mx1 * my1
            cr0 = wx * my0
            cr1 = -(wx * my1)
            cc0 = mx0 * wy
            cc1 = -(mx1 * wy)
            ccc = wx * wy

            jy0 = jnp.clip(iy0, 0, Wn - 1)
            jy1 = jnp.clip(iy1, 0, Wn - 1)
            jx0 = jnp.clip(ix0, 0, Hn - 1)
            jx1 = jnp.clip(ix1, 0, Hn - 1)

            for ch in range(C):
                chv = jnp.full((L,), ch, jnp.int32)
                acc = cornerv[ch] * ccc
                acc = acc + c00 * plsc.load_gather(b00, [q16, chv])
                acc = acc + c01 * plsc.load_gather(b01, [q16, chv])
                acc = acc + c10 * plsc.load_gather(b10, [q16, chv])
                acc = acc + c11 * plsc.load_gather(b11, [q16, chv])
                acc = acc + cr0 * plsc.load_gather(lrv, [jy0, chv])
                acc = acc + cr1 * plsc.load_gather(lrv, [jy1, chv])
                acc = acc + cc0 * plsc.load_gather(lcv, [jx0, chv])
                acc = acc + cc1 * plsc.load_gather(lcv, [jx1, chv])
                plsc.store_scatter(obuf, [q16, chv], acc)
            return carry2

        lax.fori_loop(0, STEPS, stepB, 0)

    gdescs = [None] * NCH
    xdescs = [None] * NCH
    odescs = [None] * NCH

    pltpu.sync_copy(x_hbm.at[pl.ds(base, CH)], xb[0])
    if NCH > 1:
        xdescs[1] = pltpu.async_copy(
            x_hbm.at[pl.ds(base + CH, CH)], xb[1], xsem[1])
    phase_a(0)
    gdescs[0] = fire_gathers(0, 0)

    for cc in range(1, NCH):
        p = cc & 1
        if cc + 1 < NCH:
            xdescs[cc + 1] = pltpu.async_copy(
                x_hbm.at[pl.ds(base + (cc + 1) * CH, CH)], xb[1 - p],
                xsem[1 - p])
        xdescs[cc].wait()
        phase_a(p)
        gdescs[cc] = fire_gathers(cc, p)

        for dsc in gdescs[cc - 1]:
            dsc.wait()
        if cc - 3 >= 0:
            odescs[cc - 3].wait()
        phase_b(1 - p)
        odescs[cc - 1] = pltpu.async_copy(
            ob[1 - p], out_hbm.at[pl.ds(base + (cc - 1) * CH, CH)],
            osem[1 - p])

    pl_ = (NCH - 1) & 1
    for dsc in gdescs[NCH - 1]:
        dsc.wait()
    if NCH - 3 >= 0:
        odescs[NCH - 3].wait()
    phase_b(pl_)
    odescs[NCH - 1] = pltpu.async_copy(
        ob[pl_], out_hbm.at[pl.ds(base + (NCH - 1) * CH, CH)], osem[pl_])
    if NCH - 2 >= 0:
        odescs[NCH - 2].wait()
    odescs[NCH - 1].wait()


def kernel(x, sat):
    N = x.shape[0]
    Hn, Wn, C = sat.shape
    QW = N // NW

    satf = _relayout(sat)

    mesh = plsc.VectorSubcoreMesh(core_axis_name="c", subcore_axis_name="s",
                                  num_cores=NC, num_subcores=NS)
    body = functools.partial(_sc_body, Hn, Wn, C, QW)
    fn = pl.kernel(
        body,
        out_type=jax.ShapeDtypeStruct((N, C), jnp.float32),
        mesh=mesh,
        compiler_params=pltpu.CompilerParams(needs_layout_passes=False,
                                             use_tc_tiling_on_sc=False),
        scratch_types=[
            pltpu.VMEM((CH, 4), jnp.float32),
            pltpu.VMEM((CH, 4), jnp.float32),
            pltpu.VMEM((NSUB, SUB), jnp.int32),
            pltpu.VMEM((NSUB, SUB), jnp.int32),
            pltpu.VMEM((NSUB, SUB), jnp.int32),
            pltpu.VMEM((NSUB, SUB), jnp.int32),
            pltpu.VMEM((NSUB, SUB), jnp.int32),
            pltpu.VMEM((NSUB, SUB), jnp.int32),
            pltpu.VMEM((NSUB, SUB), jnp.int32),
            pltpu.VMEM((NSUB, SUB), jnp.int32),
            pltpu.VMEM((4, CH), jnp.int32),
            pltpu.VMEM((4, CH), jnp.int32),
            pltpu.VMEM((2, CH), jnp.float32),
            pltpu.VMEM((2, CH), jnp.float32),
            pltpu.VMEM((CH, C), jnp.float32),
            pltpu.VMEM((CH, C), jnp.float32),
            pltpu.VMEM((CH, C), jnp.float32),
            pltpu.VMEM((CH, C), jnp.float32),
            pltpu.VMEM((CH, C), jnp.float32),
            pltpu.VMEM((CH, C), jnp.float32),
            pltpu.VMEM((CH, C), jnp.float32),
            pltpu.VMEM((CH, C), jnp.float32),
            pltpu.VMEM((CH, C), jnp.float32),
            pltpu.VMEM((CH, C), jnp.float32),
            pltpu.VMEM((Wn, C), jnp.float32),
            pltpu.VMEM((Hn, C), jnp.float32),
            pltpu.VMEM((Hn // SUB, SUB), jnp.int32),
            pltpu.VMEM((C, L), jnp.float32),
            pltpu.SemaphoreType.DMA,
            pltpu.SemaphoreType.DMA,
            pltpu.SemaphoreType.DMA,
            pltpu.SemaphoreType.DMA,
            pltpu.SemaphoreType.DMA,
            pltpu.SemaphoreType.DMA,
        ],
    )
    return fn(x, satf)


# speedup vs baseline: 5.1310x; 1.1130x over previous
import functools

import jax
import jax.numpy as jnp
from jax import lax
from jax.experimental import pallas as pl
from jax.experimental.pallas import tpu as pltpu, tpu_sc as plsc

NC, NS, L = 2, 16, 16
NW = NC * NS

CH = 512
SUB = 128
NSUB = CH // SUB
STEPS = CH // L
RB = 32


def _ffloor(z):
    zi = z.astype(jnp.int32)
    zf = zi.astype(jnp.float32)
    return jnp.where(zf > z, zf - 1.0, zf)


def _ifloor(z):
    zi = z.astype(jnp.int32)
    return jnp.where(zi.astype(jnp.float32) > z, zi - 1, zi)


def _relayout_body(x_ref, o_ref):
    o_ref[...] = x_ref[...].reshape(o_ref.shape)


def _relayout(sat):
    Hn, Wn, C = sat.shape
    satw = sat.reshape(Hn, Wn * C)
    kb = RB * Wn * C // 1024
    outp = pl.pallas_call(
        _relayout_body,
        grid=(Hn // RB,),
        in_specs=[pl.BlockSpec((RB, Wn * C), lambda g: (g, 0))],
        out_specs=pl.BlockSpec((kb, 8, 128), lambda g: (g, 0, 0)),
        out_shape=jax.ShapeDtypeStruct((Hn * Wn * C // 1024, 8, 128),
                                       jnp.float32),
    )(satw)
    return outp.reshape(Hn * Wn, C)


def _sc_body(Hn, Wn, C, QW,
             x_hbm, satf_hbm, out_hbm,
             xb0, xb1,
             i00a, i01a, i10a, i11a, i00b, i01b, i10b, i11b,
             sta, stb, swa, swb,
             g00a, g01a, g10a, g11a, g00b, g01b, g10b, g11b,
             ob0, ob1, lrv, lcv, lcidx, cornerv,
             gsem0, gsem1, xsem0, xsem1, osem0, osem1):
    wid = lax.axis_index("s") * NC + lax.axis_index("c")
    base = wid * QW
    NCH = QW // CH

    xb = (xb0, xb1)
    idx = ((i00a, i01a, i10a, i11a), (i00b, i01b, i10b, i11b))
    sti = (sta, stb)
    stw = (swa, swb)
    gb = ((g00a, g01a, g10a, g11a), (g00b, g01b, g10b, g11b))
    ob = (ob0, ob1)
    gsem = (gsem0, gsem1)
    xsem = (xsem0, xsem1)
    osem = (osem0, osem1)

    iota = lax.iota(jnp.int32, L)
    fH = float(Hn)
    fW = float(Wn)
    lim = Hn * Wn - 1

    pltpu.sync_copy(satf_hbm.at[pl.ds((Hn - 1) * Wn, Wn)], lrv)

    def lcidx_step(s, carry):
        vec = (s * L + iota) * Wn + (Wn - 1)
        lcidx[s // (SUB // L), pl.ds((s % (SUB // L)) * L, L)] = vec
        return carry

    lax.fori_loop(0, Hn // L, lcidx_step, 0)
    lcdescs = [pltpu.async_copy(satf_hbm.at[lcidx.at[j]],
                                lcv.at[pl.ds(j * SUB, SUB)], gsem0)
               for j in range(Hn // SUB)]
    for dsc in lcdescs:
        dsc.wait()
    for ch in range(C):
        cornerv[ch] = plsc.load_gather(
            lrv, [jnp.full((L,), Wn - 1, jnp.int32),
                  jnp.full((L,), ch, jnp.int32)])

    def phase_a(p):
        xbuf = xb[p]
        i0, i1, i2, i3 = idx[p]
        st = sti[p]
        sw = stw[p]

        def stepA(s, carry2):
            q16 = s * L + iota
            z0 = jnp.zeros((L,), jnp.int32)
            cu = plsc.load_gather(xbuf, [q16, z0])
            cv = plsc.load_gather(xbuf, [q16, z0 + 1])
            d0 = plsc.load_gather(xbuf, [q16, z0 + 2])
            d1 = plsc.load_gather(xbuf, [q16, z0 + 3])
            s0 = cu - d0 * 0.5
            e0 = cu + d0 * 0.5
            s1 = cv - d1 * 0.5
            e1 = cv + d1 * 0.5
            ns0 = jnp.minimum(s0, e0)
            ne0 = jnp.maximum(s0, e0)
            ns1 = jnp.minimum(s1, e1)
            ne1 = jnp.maximum(s1, e1)
            ns0 = ns0 - _ffloor(ns0)
            ne0 = ne0 - _ffloor(ne0)
            ns1 = ns1 - _ffloor(ns1)
            ne1 = ne1 - _ffloor(ne1)
            ix0 = _ifloor(fH * ne0 - 0.5)
            ix1 = _ifloor(fH * ns0 - 1.5)
            iy0 = _ifloor(fW * ne1 - 0.5)
            iy1 = _ifloor(fW * ns1 - 1.5)
            wx = jnp.where(ns0 > ne0, 1.0, 0.0)
            wy = jnp.where(ns1 > ne1, 1.0, 0.0)

            f00 = jnp.clip(ix0 * Wn + iy0, 0, lim)
            f01 = jnp.clip(ix0 * Wn + iy1, 0, lim)
            f10 = jnp.clip(ix1 * Wn + iy0, 0, lim)
            f11 = jnp.clip(ix1 * Wn + iy1, 0, lim)

            blk = s // (SUB // L)
            off = (s % (SUB // L)) * L
            i0[blk, pl.ds(off, L)] = f00
            i1[blk, pl.ds(off, L)] = f01
            i2[blk, pl.ds(off, L)] = f10
            i3[blk, pl.ds(off, L)] = f11

            sl = pl.ds(s * L, L)
            st[0, sl] = ix0
            st[1, sl] = ix1
            st[2, sl] = iy0
            st[3, sl] = iy1
            sw[0, sl] = wx
            sw[1, sl] = wy
            return carry2

        lax.fori_loop(0, STEPS, stepA, 0)

    def fire_gathers(cc, p):
        descs = []
        for idxr, gbuf in zip(idx[p], gb[p]):
            for j in range(NSUB):
                descs.append(pltpu.async_copy(
                    satf_hbm.at[idxr.at[j]], gbuf.at[pl.ds(j * SUB, SUB)],
                    gsem[p]))
        return descs

    def phase_b(p):
        b00, b01, b10, b11 = gb[p]
        st = sti[p]
        sw = stw[p]
        obuf = ob[p]

        def stepB(s, carry2):
            q16 = s * L + iota
            sl = pl.ds(s * L, L)
            ix0 = st[0, sl]
            ix1 = st[1, sl]
            iy0 = st[2, sl]
            iy1 = st[3, sl]
            wx = sw[0, sl]
            wy = sw[1, sl]

            one = jnp.ones((L,), jnp.float32)
            zero = jnp.zeros((L,), jnp.float32)
            mx0 = jnp.where((ix0 >= 0) & (ix0 < Hn), one, zero)
            mx1 = jnp.where((ix1 >= 0) & (ix1 < Hn), one, zero)
            my0 = jnp.where((iy0 >= 0) & (iy0 < Wn), one, zero)
            my1 = jnp.where((iy1 >= 0) & (iy1 < Wn), one, zero)

            c00 = mx0 * my0
            c01 = -(mx0 * my1)
            c10 = -(mx1 * my0)
            c11 = mx1 * my1
            cr0 = wx * my0
            cr1 = -(wx * my1)
            cc0 = mx0 * wy
            cc1 = -(mx1 * wy)
            ccc = wx * wy

            jy0 = jnp.clip(iy0, 0, Wn - 1)
            jy1 = jnp.clip(iy1, 0, Wn - 1)
            jx0 = jnp.clip(ix0, 0, Hn - 1)
            jx1 = jnp.clip(ix1, 0, Hn - 1)

            for ch in range(C):
                chv = jnp.full((L,), ch, jnp.int32)
                acc = cornerv[ch] * ccc
                acc = acc + c00 * plsc.load_gather(b00, [q16, chv])
                acc = acc + c01 * plsc.load_gather(b01, [q16, chv])
                acc = acc + c10 * plsc.load_gather(b10, [q16, chv])
                acc = acc + c11 * plsc.load_gather(b11, [q16, chv])
                acc = acc + cr0 * plsc.load_gather(lrv, [jy0, chv])
                acc = acc + cr1 * plsc.load_gather(lrv, [jy1, chv])
                acc = acc + cc0 * plsc.load_gather(lcv, [jx0, chv])
                acc = acc + cc1 * plsc.load_gather(lcv, [jx1, chv])
                plsc.store_scatter(obuf, [q16, chv], acc)
            return carry2

        lax.fori_loop(0, STEPS, stepB, 0)

    gdescs = [None] * NCH
    xdescs = [None] * NCH
    odescs = [None] * NCH

    pltpu.sync_copy(x_hbm.at[pl.ds(base, CH)], xb[0])
    if NCH > 1:
        xdescs[1] = pltpu.async_copy(
            x_hbm.at[pl.ds(base + CH, CH)], xb[1], xsem[1])
    phase_a(0)
    gdescs[0] = fire_gathers(0, 0)

    for cc in range(1, NCH):
        p = cc & 1
        if cc + 1 < NCH:
            xdescs[cc + 1] = pltpu.async_copy(
                x_hbm.at[pl.ds(base + (cc + 1) * CH, CH)], xb[1 - p],
                xsem[1 - p])
        xdescs[cc].wait()
        phase_a(p)
        gdescs[cc] = fire_gathers(cc, p)

        for dsc in gdescs[cc - 1]:
            dsc.wait()
        if cc - 3 >= 0:
            odescs[cc - 3].wait()
        phase_b(1 - p)
        odescs[cc - 1] = pltpu.async_copy(
            ob[1 - p], out_hbm.at[pl.ds(base + (cc - 1) * CH, CH)],
            osem[1 - p])

    pl_ = (NCH - 1) & 1
    for dsc in gdescs[NCH - 1]:
        dsc.wait()
    if NCH - 3 >= 0:
        odescs[NCH - 3].wait()
    phase_b(pl_)
    odescs[NCH - 1] = pltpu.async_copy(
        ob[pl_], out_hbm.at[pl.ds(base + (NCH - 1) * CH, CH)], osem[pl_])
    if NCH - 2 >= 0:
        odescs[NCH - 2].wait()
    odescs[NCH - 1].wait()


def kernel(x, sat):
    N = x.shape[0]
    Hn, Wn, C = sat.shape
    QW = N // NW

    satf = _relayout(sat)

    mesh = plsc.VectorSubcoreMesh(core_axis_name="c", subcore_axis_name="s",
                                  num_cores=NC, num_subcores=NS)
    body = functools.partial(_sc_body, Hn, Wn, C, QW)
    fn = pl.kernel(
        body,
        out_type=jax.ShapeDtypeStruct((N, C), jnp.float32),
        mesh=mesh,
        compiler_params=pltpu.CompilerParams(needs_layout_passes=False,
                                             use_tc_tiling_on_sc=False),
        scratch_types=[
            pltpu.VMEM((CH, 4), jnp.float32),
            pltpu.VMEM((CH, 4), jnp.float32),
            pltpu.VMEM((NSUB, SUB), jnp.int32),
            pltpu.VMEM((NSUB, SUB), jnp.int32),
            pltpu.VMEM((NSUB, SUB), jnp.int32),
            pltpu.VMEM((NSUB, SUB), jnp.int32),
            pltpu.VMEM((NSUB, SUB), jnp.int32),
            pltpu.VMEM((NSUB, SUB), jnp.int32),
            pltpu.VMEM((NSUB, SUB), jnp.int32),
            pltpu.VMEM((NSUB, SUB), jnp.int32),
            pltpu.VMEM((4, CH), jnp.int32),
            pltpu.VMEM((4, CH), jnp.int32),
            pltpu.VMEM((2, CH), jnp.float32),
            pltpu.VMEM((2, CH), jnp.float32),
            pltpu.VMEM((CH, C), jnp.float32),
            pltpu.VMEM((CH, C), jnp.float32),
            pltpu.VMEM((CH, C), jnp.float32),
            pltpu.VMEM((CH, C), jnp.float32),
            pltpu.VMEM((CH, C), jnp.float32),
            pltpu.VMEM((CH, C), jnp.float32),
            pltpu.VMEM((CH, C), jnp.float32),
            pltpu.VMEM((CH, C), jnp.float32),
            pltpu.VMEM((CH, C), jnp.float32),
            pltpu.VMEM((CH, C), jnp.float32),
            pltpu.VMEM((Wn, C), jnp.float32),
            pltpu.VMEM((Hn, C), jnp.float32),
            pltpu.VMEM((Hn // SUB, SUB), jnp.int32),
            pltpu.VMEM((C, L), jnp.float32),
            pltpu.SemaphoreType.DMA,
            pltpu.SemaphoreType.DMA,
            pltpu.SemaphoreType.DMA,
            pltpu.SemaphoreType.DMA,
            pltpu.SemaphoreType.DMA,
            pltpu.SemaphoreType.DMA,
        ],
    )
    return fn(x, satf)


# speedup vs baseline: 5.1974x; 1.0129x over previous
import functools

import jax
import jax.numpy as jnp
from jax import lax
from jax.experimental import pallas as pl
from jax.experimental.pallas import tpu as pltpu, tpu_sc as plsc

NC, NS, L = 2, 16, 16
NW = NC * NS

CH = 512
SUB = 128
NSUB = CH // SUB
STEPS = CH // L
RB = 64


def _ffloor(z):
    zi = z.astype(jnp.int32)
    zf = zi.astype(jnp.float32)
    return jnp.where(zf > z, zf - 1.0, zf)


def _ifloor(z):
    zi = z.astype(jnp.int32)
    return jnp.where(zi.astype(jnp.float32) > z, zi - 1, zi)


def _relayout_body(x_ref, o_ref):
    o_ref[...] = x_ref[...].reshape(o_ref.shape)


def _relayout(sat):
    Hn, Wn, C = sat.shape
    satw = sat.reshape(Hn, Wn * C)
    kb = RB * Wn * C // 1024
    outp = pl.pallas_call(
        _relayout_body,
        grid=(Hn // RB,),
        in_specs=[pl.BlockSpec((RB, Wn * C), lambda g: (g, 0))],
        out_specs=pl.BlockSpec((kb, 8, 128), lambda g: (g, 0, 0)),
        out_shape=jax.ShapeDtypeStruct((Hn * Wn * C // 1024, 8, 128),
                                       jnp.float32),
    )(satw)
    return outp.reshape(Hn * Wn, C)


def _sc_body(Hn, Wn, C, QW,
             x_hbm, satf_hbm, out_hbm,
             xb0, xb1,
             i00a, i01a, i10a, i11a, i00b, i01b, i10b, i11b,
             sta, stb, swa, swb,
             g00a, g01a, g10a, g11a, g00b, g01b, g10b, g11b,
             ob0, ob1, lrv, lcv, lcidx, cornerv,
             gsem0, gsem1, xsem0, xsem1, osem0, osem1):
    wid = lax.axis_index("s") * NC + lax.axis_index("c")
    base = wid * QW
    NCH = QW // CH

    xb = (xb0, xb1)
    idx = ((i00a, i01a, i10a, i11a), (i00b, i01b, i10b, i11b))
    sti = (sta, stb)
    stw = (swa, swb)
    gb = ((g00a, g01a, g10a, g11a), (g00b, g01b, g10b, g11b))
    ob = (ob0, ob1)
    gsem = (gsem0, gsem1)
    xsem = (xsem0, xsem1)
    osem = (osem0, osem1)

    iota = lax.iota(jnp.int32, L)
    fH = float(Hn)
    fW = float(Wn)
    lim = Hn * Wn - 1

    pltpu.sync_copy(satf_hbm.at[pl.ds((Hn - 1) * Wn, Wn)], lrv)

    def lcidx_step(s, carry):
        vec = (s * L + iota) * Wn + (Wn - 1)
        lcidx[s // (SUB // L), pl.ds((s % (SUB // L)) * L, L)] = vec
        return carry

    lax.fori_loop(0, Hn // L, lcidx_step, 0)
    lcdescs = [pltpu.async_copy(satf_hbm.at[lcidx.at[j]],
                                lcv.at[pl.ds(j * SUB, SUB)], gsem0)
               for j in range(Hn // SUB)]
    for dsc in lcdescs:
        dsc.wait()
    for ch in range(C):
        cornerv[ch] = plsc.load_gather(
            lrv, [jnp.full((L,), Wn - 1, jnp.int32),
                  jnp.full((L,), ch, jnp.int32)])

    def phase_a(p):
        xbuf = xb[p]
        i0, i1, i2, i3 = idx[p]
        st = sti[p]
        sw = stw[p]

        def stepA(s, carry2):
            q16 = s * L + iota
            z0 = jnp.zeros((L,), jnp.int32)
            cu = plsc.load_gather(xbuf, [q16, z0])
            cv = plsc.load_gather(xbuf, [q16, z0 + 1])
            d0 = plsc.load_gather(xbuf, [q16, z0 + 2])
            d1 = plsc.load_gather(xbuf, [q16, z0 + 3])
            s0 = cu - d0 * 0.5
            e0 = cu + d0 * 0.5
            s1 = cv - d1 * 0.5
            e1 = cv + d1 * 0.5
            ns0 = jnp.minimum(s0, e0)
            ne0 = jnp.maximum(s0, e0)
            ns1 = jnp.minimum(s1, e1)
            ne1 = jnp.maximum(s1, e1)
            ns0 = ns0 - _ffloor(ns0)
            ne0 = ne0 - _ffloor(ne0)
            ns1 = ns1 - _ffloor(ns1)
            ne1 = ne1 - _ffloor(ne1)
            ix0 = _ifloor(fH * ne0 - 0.5)
            ix1 = _ifloor(fH * ns0 - 1.5)
            iy0 = _ifloor(fW * ne1 - 0.5)
            iy1 = _ifloor(fW * ns1 - 1.5)
            wx = jnp.where(ns0 > ne0, 1.0, 0.0)
            wy = jnp.where(ns1 > ne1, 1.0, 0.0)

            f00 = jnp.clip(ix0 * Wn + iy0, 0, lim)
            f01 = jnp.clip(ix0 * Wn + iy1, 0, lim)
            f10 = jnp.clip(ix1 * Wn + iy0, 0, lim)
            f11 = jnp.clip(ix1 * Wn + iy1, 0, lim)

            blk = s // (SUB // L)
            off = (s % (SUB // L)) * L
            i0[blk, pl.ds(off, L)] = f00
            i1[blk, pl.ds(off, L)] = f01
            i2[blk, pl.ds(off, L)] = f10
            i3[blk, pl.ds(off, L)] = f11

            sl = pl.ds(s * L, L)
            st[0, sl] = ix0
            st[1, sl] = ix1
            st[2, sl] = iy0
            st[3, sl] = iy1
            sw[0, sl] = wx
            sw[1, sl] = wy
            return carry2

        lax.fori_loop(0, STEPS, stepA, 0)

    def fire_gathers(cc, p):
        descs = []
        for idxr, gbuf in zip(idx[p], gb[p]):
            for j in range(NSUB):
                descs.append(pltpu.async_copy(
                    satf_hbm.at[idxr.at[j]], gbuf.at[pl.ds(j * SUB, SUB)],
                    gsem[p]))
        return descs

    def phase_b(p):
        b00, b01, b10, b11 = gb[p]
        st = sti[p]
        sw = stw[p]
        obuf = ob[p]

        def stepB(s, carry2):
            q16 = s * L + iota
            sl = pl.ds(s * L, L)
            ix0 = st[0, sl]
            ix1 = st[1, sl]
            iy0 = st[2, sl]
            iy1 = st[3, sl]
            wx = sw[0, sl]
            wy = sw[1, sl]

            one = jnp.ones((L,), jnp.float32)
            zero = jnp.zeros((L,), jnp.float32)
            mx0 = jnp.where((ix0 >= 0) & (ix0 < Hn), one, zero)
            mx1 = jnp.where((ix1 >= 0) & (ix1 < Hn), one, zero)
            my0 = jnp.where((iy0 >= 0) & (iy0 < Wn), one, zero)
            my1 = jnp.where((iy1 >= 0) & (iy1 < Wn), one, zero)

            c00 = mx0 * my0
            c01 = -(mx0 * my1)
            c10 = -(mx1 * my0)
            c11 = mx1 * my1
            cr0 = wx * my0
            cr1 = -(wx * my1)
            cc0 = mx0 * wy
            cc1 = -(mx1 * wy)
            ccc = wx * wy

            jy0 = jnp.clip(iy0, 0, Wn - 1)
            jy1 = jnp.clip(iy1, 0, Wn - 1)
            jx0 = jnp.clip(ix0, 0, Hn - 1)
            jx1 = jnp.clip(ix1, 0, Hn - 1)

            for ch in range(C):
                chv = jnp.full((L,), ch, jnp.int32)
                acc = cornerv[ch] * ccc
                acc = acc + c00 * plsc.load_gather(b00, [q16, chv])
                acc = acc + c01 * plsc.load_gather(b01, [q16, chv])
                acc = acc + c10 * plsc.load_gather(b10, [q16, chv])
                acc = acc + c11 * plsc.load_gather(b11, [q16, chv])
                acc = acc + cr0 * plsc.load_gather(lrv, [jy0, chv])
                acc = acc + cr1 * plsc.load_gather(lrv, [jy1, chv])
                acc = acc + cc0 * plsc.load_gather(lcv, [jx0, chv])
                acc = acc + cc1 * plsc.load_gather(lcv, [jx1, chv])
                plsc.store_scatter(obuf, [q16, chv], acc)
            return carry2

        lax.fori_loop(0, STEPS, stepB, 0)

    gdescs = [None] * NCH
    xdescs = [None] * NCH
    odescs = [None] * NCH

    pltpu.sync_copy(x_hbm.at[pl.ds(base, CH)], xb[0])
    if NCH > 1:
        xdescs[1] = pltpu.async_copy(
            x_hbm.at[pl.ds(base + CH, CH)], xb[1], xsem[1])
    phase_a(0)
    gdescs[0] = fire_gathers(0, 0)

    for cc in range(1, NCH):
        p = cc & 1
        if cc + 1 < NCH:
            xdescs[cc + 1] = pltpu.async_copy(
                x_hbm.at[pl.ds(base + (cc + 1) * CH, CH)], xb[1 - p],
                xsem[1 - p])
        xdescs[cc].wait()
        phase_a(p)
        gdescs[cc] = fire_gathers(cc, p)

        for dsc in gdescs[cc - 1]:
            dsc.wait()
        if cc - 3 >= 0:
            odescs[cc - 3].wait()
        phase_b(1 - p)
        odescs[cc - 1] = pltpu.async_copy(
            ob[1 - p], out_hbm.at[pl.ds(base + (cc - 1) * CH, CH)],
            osem[1 - p])

    pl_ = (NCH - 1) & 1
    for dsc in gdescs[NCH - 1]:
        dsc.wait()
    if NCH - 3 >= 0:
        odescs[NCH - 3].wait()
    phase_b(pl_)
    odescs[NCH - 1] = pltpu.async_copy(
        ob[pl_], out_hbm.at[pl.ds(base + (NCH - 1) * CH, CH)], osem[pl_])
    if NCH - 2 >= 0:
        odescs[NCH - 2].wait()
    odescs[NCH - 1].wait()


def kernel(x, sat):
    N = x.shape[0]
    Hn, Wn, C = sat.shape
    QW = N // NW

    satf = _relayout(sat)

    mesh = plsc.VectorSubcoreMesh(core_axis_name="c", subcore_axis_name="s",
                                  num_cores=NC, num_subcores=NS)
    body = functools.partial(_sc_body, Hn, Wn, C, QW)
    fn = pl.kernel(
        body,
        out_type=jax.ShapeDtypeStruct((N, C), jnp.float32),
        mesh=mesh,
        compiler_params=pltpu.CompilerParams(needs_layout_passes=False,
                                             use_tc_tiling_on_sc=False),
        scratch_types=[
            pltpu.VMEM((CH, 4), jnp.float32),
            pltpu.VMEM((CH, 4), jnp.float32),
            pltpu.VMEM((NSUB, SUB), jnp.int32),
            pltpu.VMEM((NSUB, SUB), jnp.int32),
            pltpu.VMEM((NSUB, SUB), jnp.int32),
            pltpu.VMEM((NSUB, SUB), jnp.int32),
            pltpu.VMEM((NSUB, SUB), jnp.int32),
            pltpu.VMEM((NSUB, SUB), jnp.int32),
            pltpu.VMEM((NSUB, SUB), jnp.int32),
            pltpu.VMEM((NSUB, SUB), jnp.int32),
            pltpu.VMEM((4, CH), jnp.int32),
            pltpu.VMEM((4, CH), jnp.int32),
            pltpu.VMEM((2, CH), jnp.float32),
            pltpu.VMEM((2, CH), jnp.float32),
            pltpu.VMEM((CH, C), jnp.float32),
            pltpu.VMEM((CH, C), jnp.float32),
            pltpu.VMEM((CH, C), jnp.float32),
            pltpu.VMEM((CH, C), jnp.float32),
            pltpu.VMEM((CH, C), jnp.float32),
            pltpu.VMEM((CH, C), jnp.float32),
            pltpu.VMEM((CH, C), jnp.float32),
            pltpu.VMEM((CH, C), jnp.float32),
            pltpu.VMEM((CH, C), jnp.float32),
            pltpu.VMEM((CH, C), jnp.float32),
            pltpu.VMEM((Wn, C), jnp.float32),
            pltpu.VMEM((Hn, C), jnp.float32),
            pltpu.VMEM((Hn // SUB, SUB), jnp.int32),
            pltpu.VMEM((C, L), jnp.float32),
            pltpu.SemaphoreType.DMA,
            pltpu.SemaphoreType.DMA,
            pltpu.SemaphoreType.DMA,
            pltpu.SemaphoreType.DMA,
            pltpu.SemaphoreType.DMA,
            pltpu.SemaphoreType.DMA,
        ],
    )
    return fn(x, satf)
